# Initial kernel scaffold; baseline (speedup 1.0000x reference)
import jax
import jax.numpy as jnp
from jax.experimental import pallas as pl


def kernel(x, W_w, W_b, U_w, U_b, V_w, V_b, g_w, g_b):
    raise NotImplementedError("write your pallas kernel here")


# speedup vs baseline: 1.5094x; 1.5094x over previous
import functools

import jax
import jax.numpy as jnp
from jax.experimental import pallas as pl
from jax.experimental.pallas import tpu as pltpu

N = 16384
IN_DIM = 2048
HIDDEN = 1024
E = 16
TOPK = 2
DT = 0.02

BM = 512


def _router_kernel(x_ref, u_ref, ub_ref, g_ref, gb_ref,
                   w_ref, i_ref, p_ref):
    acc = jax.lax.dot_general(
        x_ref[...], u_ref[...],
        dimension_numbers=(((1,), (1,)), ((), ())),
        preferred_element_type=jnp.float32,
        precision=jax.lax.Precision.DEFAULT,
    )
    h = DT * jnp.tanh(acc + ub_ref[...])
    logits = jax.lax.dot_general(
        h, g_ref[...],
        dimension_numbers=(((1,), (1,)), ((), ())),
        preferred_element_type=jnp.float32,
        precision=jax.lax.Precision.DEFAULT,
    ) + gb_ref[...]
    m = jnp.max(logits, axis=-1, keepdims=True)
    e = jnp.exp(logits - m)
    probs = e / jnp.sum(e, axis=-1, keepdims=True)
    p_ref[...] = probs

    col = jax.lax.broadcasted_iota(jnp.int32, probs.shape, 1)
    p1 = jnp.max(probs, axis=-1, keepdims=True)
    is1 = probs == p1
    i1 = jnp.min(jnp.where(is1, col, E), axis=-1, keepdims=True)
    masked = jnp.where(col == i1, -jnp.inf, probs)
    p2 = jnp.max(masked, axis=-1, keepdims=True)
    i2 = jnp.min(jnp.where(masked == p2, col, E), axis=-1, keepdims=True)
    denom = p1 + p2 + 1e-08
    w_ref[...] = jnp.concatenate([p1 / denom, p2 / denom], axis=1)
    i_ref[...] = jnp.concatenate([i1, i2], axis=1)


def kernel(x, W_w, W_b, U_w, U_b, V_w, V_b, g_w, g_b):
    del W_w, V_w, V_b
    bias = (W_b + U_b).reshape(1, HIDDEN)
    gb = g_b.reshape(1, E)
    grid = (N // BM,)
    weights, indices, probs = pl.pallas_call(
        _router_kernel,
        grid=grid,
        in_specs=[
            pl.BlockSpec((BM, IN_DIM), lambda i: (i, 0)),
            pl.BlockSpec((HIDDEN, IN_DIM), lambda i: (0, 0)),
            pl.BlockSpec((1, HIDDEN), lambda i: (0, 0)),
            pl.BlockSpec((E, HIDDEN), lambda i: (0, 0)),
            pl.BlockSpec((1, E), lambda i: (0, 0)),
        ],
        out_specs=[
            pl.BlockSpec((BM, TOPK), lambda i: (i, 0)),
            pl.BlockSpec((BM, TOPK), lambda i: (i, 0)),
            pl.BlockSpec((BM, E), lambda i: (i, 0)),
        ],
        out_shape=[
            jax.ShapeDtypeStruct((N, TOPK), jnp.float32),
            jax.ShapeDtypeStruct((N, TOPK), jnp.int32),
            jax.ShapeDtypeStruct((N, E), jnp.float32),
        ],
    )(x, U_w, bias, g_w, gb)
    return weights, indices, probs
